# Initial kernel scaffold; baseline (speedup 1.0000x reference)
import jax
import jax.numpy as jnp
from jax.experimental import pallas as pl


def kernel(features, W_vis, prototypes):
    raise NotImplementedError("write your pallas kernel here")


# speedup vs baseline: 15.2615x; 15.2615x over previous
import functools
import jax
import jax.numpy as jnp
from jax.experimental import pallas as pl

_N_PROTO = 8192
_FEAT = 1024
_SHARED = 256
_TOPK = 32
_TEMP = 0.1
_BATCH = 8192
_BLK_R = 256


def _body(f_ref, w_ref, p_ref, o_ref):
    f = f_ref[...]
    w = w_ref[...]
    proj = jax.lax.dot_general(
        f, w, (((1,), (1,)), ((), ())),
        preferred_element_type=jnp.float32)
    pn = jnp.sqrt(jnp.sum(proj * proj, axis=1, keepdims=True))
    proj = proj / jnp.maximum(pn, 1e-12)

    p = p_ref[...]
    sims = jax.lax.dot_general(
        proj, p, (((1,), (1,)), ((), ())),
        preferred_element_type=jnp.float32)

    neg = jnp.float32(-jnp.inf)
    m0 = jnp.max(sims, axis=1, keepdims=True)
    m = m0
    for _ in range(_TOPK - 1):
        m = jnp.max(jnp.where(sims < m, sims, neg), axis=1, keepdims=True)
    act = jnp.where(sims >= m, jnp.exp((sims - m0) * (1.0 / _TEMP)), 0.0)
    denom = jnp.sum(act, axis=1, keepdims=True)
    emb = jax.lax.dot_general(
        act, p, (((1,), (0,)), ((), ())),
        preferred_element_type=jnp.float32)
    emb = emb / denom
    en = jnp.sqrt(jnp.sum(emb * emb, axis=1, keepdims=True))
    o_ref[...] = emb / jnp.maximum(en, 1e-12)


@functools.partial(jax.jit, static_argnames=("interpret",))
def kernel(features, W_vis, prototypes, interpret=False):
    grid = (_BATCH // _BLK_R,)
    return pl.pallas_call(
        _body,
        grid=grid,
        in_specs=[
            pl.BlockSpec((_BLK_R, _FEAT), lambda i: (i, 0)),
            pl.BlockSpec((_SHARED, _FEAT), lambda i: (0, 0)),
            pl.BlockSpec((_N_PROTO, _SHARED), lambda i: (0, 0)),
        ],
        out_specs=pl.BlockSpec((_BLK_R, _SHARED), lambda i: (i, 0)),
        out_shape=jax.ShapeDtypeStruct((_BATCH, _SHARED), jnp.float32),
        interpret=interpret,
    )(features, W_vis, prototypes)


# speedup vs baseline: 15.2677x; 1.0004x over previous
import functools
import jax
import jax.numpy as jnp
from jax.experimental import pallas as pl

_N_PROTO = 8192
_FEAT = 1024
_SHARED = 256
_TOPK = 32
_TEMP = 0.1
_BATCH = 8192
_BLK_R = 256


def _body(f_ref, w_ref, p_ref, o_ref):
    f = f_ref[...]
    w = w_ref[...]
    proj = jax.lax.dot_general(
        f, w, (((1,), (1,)), ((), ())),
        preferred_element_type=jnp.float32)
    pn = jnp.sqrt(jnp.sum(proj * proj, axis=1, keepdims=True))
    proj = proj / jnp.maximum(pn, 1e-12)

    p = p_ref[...]
    sims = jax.lax.dot_general(
        proj, p, (((1,), (1,)), ((), ())),
        preferred_element_type=jnp.float32)

    neg = jnp.float32(-jnp.inf)
    m0 = jnp.max(sims, axis=1, keepdims=True)
    m = m0
    for _ in range(_TOPK - 1):
        m = jnp.max(jnp.where(sims < m, sims, neg), axis=1, keepdims=True)
    act = jnp.where(sims >= m, jnp.exp((sims - m0) * (1.0 / _TEMP)), 0.0)
    denom = jnp.sum(act, axis=1, keepdims=True)
    emb = jax.lax.dot_general(
        act, p, (((1,), (0,)), ((), ())),
        precision=jax.lax.Precision.DEFAULT,
        preferred_element_type=jnp.float32)
    emb = emb / denom
    en = jnp.sqrt(jnp.sum(emb * emb, axis=1, keepdims=True))
    o_ref[...] = emb / jnp.maximum(en, 1e-12)


@functools.partial(jax.jit, static_argnames=("interpret",))
def kernel(features, W_vis, prototypes, interpret=False):
    grid = (_BATCH // _BLK_R,)
    return pl.pallas_call(
        _body,
        grid=grid,
        in_specs=[
            pl.BlockSpec((_BLK_R, _FEAT), lambda i: (i, 0)),
            pl.BlockSpec((_SHARED, _FEAT), lambda i: (0, 0)),
            pl.BlockSpec((_N_PROTO, _SHARED), lambda i: (0, 0)),
        ],
        out_specs=pl.BlockSpec((_BLK_R, _SHARED), lambda i: (i, 0)),
        out_shape=jax.ShapeDtypeStruct((_BATCH, _SHARED), jnp.float32),
        interpret=interpret,
    )(features, W_vis, prototypes)
